# Initial kernel scaffold; baseline (speedup 1.0000x reference)
import jax
import jax.numpy as jnp
from jax.experimental import pallas as pl


def kernel(observed_trajectory_target, observed_trajectory_others, neighbor_mask, W_ih, b_ih, W_hh, b_hh, W1, b1, W2, b2, Wc, bc):
    raise NotImplementedError("write your pallas kernel here")


# speedup vs baseline: 10.7039x; 10.7039x over previous
import jax
import jax.numpy as jnp
from jax.experimental import pallas as pl
from jax.experimental.pallas import tpu as pltpu

_OBS, _N, _IN, _H = 15, 50000, 2, 64
_GX, _GY = 4, 4
_NS = 4.0
_HALF = _NS / 2.0
_INV_CW = float(_GX) / _NS
_INV_CH = float(_GY) / _NS
_HX, _HY = _GX // 2, _GY // 2

_BN = 512
_NPAD = 50176
_NB = _NPAD // _BN


def _neighbor_kernel(xo_ref, mask_ref, tgt14_ref, wcat_ref, out_ref,
                     hc_scr, c_scr):
    hc_scr[...] = jnp.zeros_like(hc_scr)
    hc_scr[2:3, :] = jnp.ones((1, _BN), jnp.float32)
    c_scr[...] = jnp.zeros_like(c_scr)

    def step(t, carry):
        xt = xo_ref[t]
        hc_scr[0:2, :] = xt
        gates = jnp.dot(wcat_ref[...], hc_scr[...],
                        preferred_element_type=jnp.float32)
        i = jax.nn.sigmoid(gates[0:64, :])
        f = jax.nn.sigmoid(gates[64:128, :])
        g = jnp.tanh(gates[128:192, :])
        o = jax.nn.sigmoid(gates[192:256, :])
        c_new = f * c_scr[...] + i * g
        c_scr[...] = c_new
        hc_scr[64:128, :] = o * jnp.tanh(c_new)
        return carry

    jax.lax.fori_loop(0, _OBS, step, 0)

    h = hc_scr[64:128, :]
    x14 = xo_ref[_OBS - 1]
    relx = x14[0:1, :] - tgt14_ref[0, 0]
    rely = x14[1:2, :] - tgt14_ref[0, 1]
    cx = jnp.trunc(relx * _INV_CW).astype(jnp.int32) + _HX
    cy = jnp.trunc(rely * _INV_CH).astype(jnp.int32) + _HY
    within = (jnp.abs(relx) <= _HALF) & (jnp.abs(rely) <= _HALF)
    valid = (within & (cx >= 0) & (cx < _GX) & (cy >= 0) & (cy < _GY)
             & (mask_ref[...] != 0))
    idx = cy * _GX + cx
    cell = jax.lax.broadcasted_iota(jnp.int32, (_GX * _GY, _BN), 0)
    onehot = jnp.where((idx == cell) & valid, 1.0, 0.0)
    out_ref[0] = jax.lax.dot_general(
        onehot, h, (((1,), (1,)), ((), ())),
        preferred_element_type=jnp.float32)


def _finish_kernel(p_ref, tgt_ref, wihT_ref, whhT_ref, b_ref,
                   w1t3_ref, b1_ref, w2T_ref, b2_ref, wcT_ref, bc_ref,
                   out_ref):
    social = jnp.sum(p_ref[...], axis=0)
    h = jnp.zeros((1, _H), jnp.float32)
    c = jnp.zeros((1, _H), jnp.float32)
    for t in range(_OBS):
        xt = tgt_ref[t]
        gates = (jnp.dot(xt, wihT_ref[...], preferred_element_type=jnp.float32)
                 + jnp.dot(h, whhT_ref[...], preferred_element_type=jnp.float32)
                 + b_ref[...])
        i = jax.nn.sigmoid(gates[:, 0:64])
        f = jax.nn.sigmoid(gates[:, 64:128])
        g = jnp.tanh(gates[:, 128:192])
        o = jax.nn.sigmoid(gates[:, 192:256])
        c = f * c + i * g
        h = o * jnp.tanh(c)
    ctxp = b1_ref[...]
    for cix in range(_GX * _GY):
        ctxp = ctxp + jnp.dot(social[cix:cix + 1, :], w1t3_ref[cix],
                              preferred_element_type=jnp.float32)
    ctx = jnp.dot(jnp.maximum(ctxp, 0.0), w2T_ref[...],
                  preferred_element_type=jnp.float32) + b2_ref[...]
    out_ref[...] = jnp.dot(h + ctx, wcT_ref[...],
                           preferred_element_type=jnp.float32) + bc_ref[...]


def kernel(observed_trajectory_target, observed_trajectory_others,
           neighbor_mask, W_ih, b_ih, W_hh, b_hh, W1, b1, W2, b2, Wc, bc):
    oth = jnp.pad(observed_trajectory_others,
                  ((0, 0), (0, _NPAD - _N), (0, 0)))
    xo = jnp.transpose(oth, (0, 2, 1))
    mask14 = jnp.pad(neighbor_mask[_OBS - 1:_OBS, :],
                     ((0, 0), (0, _NPAD - _N)))
    tgt = observed_trajectory_target[:, 0, :]
    tgt14 = tgt[_OBS - 1:_OBS, :]
    bsum = (b_ih + b_hh).reshape(1, 4 * _H)
    wcat = jnp.zeros((4 * _H, 2 * _H), jnp.float32)
    wcat = wcat.at[:, 0:2].set(W_ih)
    wcat = wcat.at[:, 2].set(b_ih + b_hh)
    wcat = wcat.at[:, _H:2 * _H].set(W_hh)
    tgt3 = tgt.reshape(_OBS, 1, 2)
    w1t3 = W1.T.reshape(_GX * _GY, _H, _H)

    partials = pl.pallas_call(
        _neighbor_kernel,
        grid=(_NB,),
        in_specs=[
            pl.BlockSpec((_OBS, 2, _BN), lambda i: (0, 0, i)),
            pl.BlockSpec((1, _BN), lambda i: (0, i)),
            pl.BlockSpec((1, 2), lambda i: (0, 0)),
            pl.BlockSpec((4 * _H, 2 * _H), lambda i: (0, 0)),
        ],
        out_specs=pl.BlockSpec((1, _GX * _GY, _H), lambda i: (i, 0, 0)),
        out_shape=jax.ShapeDtypeStruct((_NB, _GX * _GY, _H), jnp.float32),
        scratch_shapes=[
            pltpu.VMEM((2 * _H, _BN), jnp.float32),
            pltpu.VMEM((_H, _BN), jnp.float32),
        ],
        compiler_params=pltpu.CompilerParams(
            dimension_semantics=("arbitrary",),
        ),
        name="social_lstm_neighbors",
    )(xo, mask14, tgt14, wcat)

    out = pl.pallas_call(
        _finish_kernel,
        out_shape=jax.ShapeDtypeStruct((1, 2), jnp.float32),
        name="social_lstm_finish",
    )(partials, tgt3, W_ih.T, W_hh.T, bsum, w1t3,
      b1.reshape(1, _H), W2.T, b2.reshape(1, _H), Wc.T, bc.reshape(1, 2))
    return out


# speedup vs baseline: 24.5863x; 2.2969x over previous
import jax
import jax.numpy as jnp
from jax.experimental import pallas as pl
from jax.experimental.pallas import tpu as pltpu

_OBS, _N, _IN, _H = 15, 50000, 2, 64
_GX, _GY = 4, 4
_NS = 4.0
_HALF = _NS / 2.0
_INV_CW = float(_GX) / _NS
_INV_CH = float(_GY) / _NS
_HX, _HY = _GX // 2, _GY // 2

_BH = 512
_NH = 2
_BN = _BH * _NH
_NPAD = 50176
_NB = _NPAD // _BN


_BF16_STEPS = 12


def _sig2(z):
    return jnp.tanh(z) * 0.5 + 0.5


def _neighbor_kernel(xo_ref, mask_ref, tgt14_ref, wcatb_ref, wcatf_ref,
                     out_ref, hcb_scr, hcf_scr, c_scr):
    hcb_scr[...] = jnp.zeros_like(hcb_scr)
    hcf_scr[...] = jnp.zeros_like(hcf_scr)
    for u in range(_NH):
        hcb_scr[u, 2:3, :] = jnp.ones((1, _BH), jnp.bfloat16)
        hcf_scr[u, 2:3, :] = jnp.ones((1, _BH), jnp.float32)
        c_scr[u] = jnp.zeros((_H, _BH), jnp.float32)

    for t in range(_OBS):
        bf = t < _BF16_STEPS
        hc_scr = hcb_scr if bf else hcf_scr
        w_ref = wcatb_ref if bf else wcatf_ref
        for u in range(_NH):
            xt = xo_ref[t][:, u * _BH:(u + 1) * _BH]
            hc_scr[u, 0:2, :] = xt.astype(hc_scr.dtype)
            gates = jnp.dot(w_ref[...], hc_scr[u],
                            preferred_element_type=jnp.float32)
            i = _sig2(gates[0:64, :])
            f = _sig2(gates[64:128, :])
            g = jnp.tanh(gates[128:192, :])
            o = _sig2(gates[192:256, :])
            c_new = f * c_scr[u] + i * g
            c_scr[u] = c_new
            h_new = o * jnp.tanh(c_new)
            if t + 1 == _BF16_STEPS:
                hcf_scr[u, 64:128, :] = h_new
            elif bf:
                hcb_scr[u, 64:128, :] = h_new.astype(jnp.bfloat16)
            else:
                hcf_scr[u, 64:128, :] = h_new

    x14 = xo_ref[_OBS - 1]
    social = jnp.zeros((_GX * _GY, _H), jnp.float32)
    for u in range(_NH):
        h = hcf_scr[u, 64:128, :]
        relx = x14[0:1, u * _BH:(u + 1) * _BH] - tgt14_ref[0, 0]
        rely = x14[1:2, u * _BH:(u + 1) * _BH] - tgt14_ref[0, 1]
        cx = jnp.trunc(relx * _INV_CW).astype(jnp.int32) + _HX
        cy = jnp.trunc(rely * _INV_CH).astype(jnp.int32) + _HY
        within = (jnp.abs(relx) <= _HALF) & (jnp.abs(rely) <= _HALF)
        valid = (within & (cx >= 0) & (cx < _GX) & (cy >= 0) & (cy < _GY)
                 & (mask_ref[:, u * _BH:(u + 1) * _BH] != 0))
        idx = cy * _GX + cx
        cell = jax.lax.broadcasted_iota(jnp.int32, (_GX * _GY, _BH), 0)
        onehot = jnp.where((idx == cell) & valid, 1.0, 0.0)
        social = social + jax.lax.dot_general(
            onehot, h, (((1,), (1,)), ((), ())),
            preferred_element_type=jnp.float32)
    out_ref[0] = social


def _finish_kernel(p_ref, tgt_ref, wihT_ref, whhT_ref, b_ref,
                   w1t3_ref, b1_ref, w2T_ref, b2_ref, wcT_ref, bc_ref,
                   out_ref):
    social = jnp.sum(p_ref[...], axis=0)
    h = jnp.zeros((1, _H), jnp.float32)
    c = jnp.zeros((1, _H), jnp.float32)
    for t in range(_OBS):
        xt = tgt_ref[t]
        gates = (jnp.dot(xt, wihT_ref[...], preferred_element_type=jnp.float32)
                 + jnp.dot(h, whhT_ref[...], preferred_element_type=jnp.float32)
                 + b_ref[...])
        i = jax.nn.sigmoid(gates[:, 0:64])
        f = jax.nn.sigmoid(gates[:, 64:128])
        g = jnp.tanh(gates[:, 128:192])
        o = jax.nn.sigmoid(gates[:, 192:256])
        c = f * c + i * g
        h = o * jnp.tanh(c)
    ctxp = b1_ref[...]
    for cix in range(_GX * _GY):
        ctxp = ctxp + jnp.dot(social[cix:cix + 1, :], w1t3_ref[cix],
                              preferred_element_type=jnp.float32)
    ctx = jnp.dot(jnp.maximum(ctxp, 0.0), w2T_ref[...],
                  preferred_element_type=jnp.float32) + b2_ref[...]
    out_ref[...] = jnp.dot(h + ctx, wcT_ref[...],
                           preferred_element_type=jnp.float32) + bc_ref[...]


def kernel(observed_trajectory_target, observed_trajectory_others,
           neighbor_mask, W_ih, b_ih, W_hh, b_hh, W1, b1, W2, b2, Wc, bc):
    oth = jnp.pad(observed_trajectory_others,
                  ((0, 0), (0, _NPAD - _N), (0, 0)))
    xo = jnp.transpose(oth, (0, 2, 1))
    mask14 = jnp.pad(neighbor_mask[_OBS - 1:_OBS, :],
                     ((0, 0), (0, _NPAD - _N)))
    tgt = observed_trajectory_target[:, 0, :]
    tgt14 = tgt[_OBS - 1:_OBS, :]
    bsum = (b_ih + b_hh).reshape(1, 4 * _H)
    wcat = jnp.zeros((4 * _H, 2 * _H), jnp.float32)
    wcat = wcat.at[:, 0:2].set(W_ih)
    wcat = wcat.at[:, 2].set(b_ih + b_hh)
    wcat = wcat.at[:, _H:2 * _H].set(W_hh)
    scale = jnp.concatenate([jnp.full((2 * _H, 1), 0.5, jnp.float32),
                             jnp.ones((_H, 1), jnp.float32),
                             jnp.full((_H, 1), 0.5, jnp.float32)])
    wcat = wcat * scale
    tgt3 = tgt.reshape(_OBS, 1, 2)
    w1t3 = W1.T.reshape(_GX * _GY, _H, _H)

    partials = pl.pallas_call(
        _neighbor_kernel,
        grid=(_NB,),
        in_specs=[
            pl.BlockSpec((_OBS, 2, _BN), lambda i: (0, 0, i)),
            pl.BlockSpec((1, _BN), lambda i: (0, i)),
            pl.BlockSpec((1, 2), lambda i: (0, 0)),
            pl.BlockSpec((4 * _H, 2 * _H), lambda i: (0, 0)),
            pl.BlockSpec((4 * _H, 2 * _H), lambda i: (0, 0)),
        ],
        out_specs=pl.BlockSpec((1, _GX * _GY, _H), lambda i: (i, 0, 0)),
        out_shape=jax.ShapeDtypeStruct((_NB, _GX * _GY, _H), jnp.float32),
        scratch_shapes=[
            pltpu.VMEM((_NH, 2 * _H, _BH), jnp.bfloat16),
            pltpu.VMEM((_NH, 2 * _H, _BH), jnp.float32),
            pltpu.VMEM((_NH, _H, _BH), jnp.float32),
        ],
        compiler_params=pltpu.CompilerParams(
            dimension_semantics=("arbitrary",),
        ),
        name="social_lstm_neighbors",
    )(xo, mask14, tgt14, wcat.astype(jnp.bfloat16), wcat)

    out = pl.pallas_call(
        _finish_kernel,
        out_shape=jax.ShapeDtypeStruct((1, 2), jnp.float32),
        name="social_lstm_finish",
    )(partials, tgt3, W_ih.T, W_hh.T, bsum, w1t3,
      b1.reshape(1, _H), W2.T, b2.reshape(1, _H), Wc.T, bc.reshape(1, 2))
    return out


# speedup vs baseline: 25.1839x; 1.0243x over previous
import jax
import jax.numpy as jnp
from jax.experimental import pallas as pl
from jax.experimental.pallas import tpu as pltpu

_OBS, _N, _IN, _H = 15, 50000, 2, 64
_GX, _GY = 4, 4
_NS = 4.0
_HALF = _NS / 2.0
_INV_CW = float(_GX) / _NS
_INV_CH = float(_GY) / _NS
_HX, _HY = _GX // 2, _GY // 2

_BH = 512
_NH = 2
_BN = _BH * _NH
_NPAD = 50176
_NB = _NPAD // _BN


_BF16_STEPS = 10


def _sig2(z):
    return jnp.tanh(z) * 0.5 + 0.5


def _neighbor_kernel(xo_ref, mask_ref, tgt14_ref, wcatb_ref, wcatf_ref,
                     out_ref, hcb_scr, hcf_scr, c_scr):
    hcb_scr[...] = jnp.zeros_like(hcb_scr)
    hcf_scr[...] = jnp.zeros_like(hcf_scr)
    for u in range(_NH):
        hcb_scr[u, 2:3, :] = jnp.ones((1, _BH), jnp.bfloat16)
        hcf_scr[u, 2:3, :] = jnp.ones((1, _BH), jnp.float32)
        c_scr[u] = jnp.zeros((_H, _BH), jnp.float32)

    for t in range(_OBS):
        bf = t < _BF16_STEPS
        hc_scr = hcb_scr if bf else hcf_scr
        w_ref = wcatb_ref if bf else wcatf_ref
        for u in range(_NH):
            xt = xo_ref[t][:, u * _BH:(u + 1) * _BH]
            hc_scr[u, 0:2, :] = xt.astype(hc_scr.dtype)
            gates = jnp.dot(w_ref[...], hc_scr[u],
                            preferred_element_type=jnp.float32)
            i = _sig2(gates[0:64, :])
            f = _sig2(gates[64:128, :])
            g = jnp.tanh(gates[128:192, :])
            o = _sig2(gates[192:256, :])
            c_new = f * c_scr[u] + i * g
            c_scr[u] = c_new
            h_new = o * jnp.tanh(c_new)
            if t + 1 == _BF16_STEPS:
                hcf_scr[u, 64:128, :] = h_new
            elif bf:
                hcb_scr[u, 64:128, :] = h_new.astype(jnp.bfloat16)
            else:
                hcf_scr[u, 64:128, :] = h_new

    x14 = xo_ref[_OBS - 1]
    social = jnp.zeros((_GX * _GY, _H), jnp.float32)
    for u in range(_NH):
        h = hcf_scr[u, 64:128, :]
        relx = x14[0:1, u * _BH:(u + 1) * _BH] - tgt14_ref[0, 0]
        rely = x14[1:2, u * _BH:(u + 1) * _BH] - tgt14_ref[0, 1]
        cx = jnp.trunc(relx * _INV_CW).astype(jnp.int32) + _HX
        cy = jnp.trunc(rely * _INV_CH).astype(jnp.int32) + _HY
        within = (jnp.abs(relx) <= _HALF) & (jnp.abs(rely) <= _HALF)
        valid = (within & (cx >= 0) & (cx < _GX) & (cy >= 0) & (cy < _GY)
                 & (mask_ref[:, u * _BH:(u + 1) * _BH] != 0))
        idx = cy * _GX + cx
        cell = jax.lax.broadcasted_iota(jnp.int32, (_GX * _GY, _BH), 0)
        onehot = jnp.where((idx == cell) & valid, 1.0, 0.0)
        social = social + jax.lax.dot_general(
            onehot, h, (((1,), (1,)), ((), ())),
            preferred_element_type=jnp.float32)
    out_ref[0] = social


def _finish_kernel(p_ref, tgt_ref, wihT_ref, whhT_ref, b_ref,
                   w1t3_ref, b1_ref, w2T_ref, b2_ref, wcT_ref, bc_ref,
                   out_ref):
    social = jnp.sum(p_ref[...], axis=0)
    h = jnp.zeros((1, _H), jnp.float32)
    c = jnp.zeros((1, _H), jnp.float32)
    for t in range(_OBS):
        xt = tgt_ref[t]
        gates = (jnp.dot(xt, wihT_ref[...], preferred_element_type=jnp.float32)
                 + jnp.dot(h, whhT_ref[...], preferred_element_type=jnp.float32)
                 + b_ref[...])
        i = jax.nn.sigmoid(gates[:, 0:64])
        f = jax.nn.sigmoid(gates[:, 64:128])
        g = jnp.tanh(gates[:, 128:192])
        o = jax.nn.sigmoid(gates[:, 192:256])
        c = f * c + i * g
        h = o * jnp.tanh(c)
    ctxp = b1_ref[...]
    for cix in range(_GX * _GY):
        ctxp = ctxp + jnp.dot(social[cix:cix + 1, :], w1t3_ref[cix],
                              preferred_element_type=jnp.float32)
    ctx = jnp.dot(jnp.maximum(ctxp, 0.0), w2T_ref[...],
                  preferred_element_type=jnp.float32) + b2_ref[...]
    out_ref[...] = jnp.dot(h + ctx, wcT_ref[...],
                           preferred_element_type=jnp.float32) + bc_ref[...]


def kernel(observed_trajectory_target, observed_trajectory_others,
           neighbor_mask, W_ih, b_ih, W_hh, b_hh, W1, b1, W2, b2, Wc, bc):
    oth = jnp.pad(observed_trajectory_others,
                  ((0, 0), (0, _NPAD - _N), (0, 0)))
    xo = jnp.transpose(oth, (0, 2, 1))
    mask14 = jnp.pad(neighbor_mask[_OBS - 1:_OBS, :],
                     ((0, 0), (0, _NPAD - _N)))
    tgt = observed_trajectory_target[:, 0, :]
    tgt14 = tgt[_OBS - 1:_OBS, :]
    bsum = (b_ih + b_hh).reshape(1, 4 * _H)
    wcat = jnp.zeros((4 * _H, 2 * _H), jnp.float32)
    wcat = wcat.at[:, 0:2].set(W_ih)
    wcat = wcat.at[:, 2].set(b_ih + b_hh)
    wcat = wcat.at[:, _H:2 * _H].set(W_hh)
    scale = jnp.concatenate([jnp.full((2 * _H, 1), 0.5, jnp.float32),
                             jnp.ones((_H, 1), jnp.float32),
                             jnp.full((_H, 1), 0.5, jnp.float32)])
    wcat = wcat * scale
    tgt3 = tgt.reshape(_OBS, 1, 2)
    w1t3 = W1.T.reshape(_GX * _GY, _H, _H)

    partials = pl.pallas_call(
        _neighbor_kernel,
        grid=(_NB,),
        in_specs=[
            pl.BlockSpec((_OBS, 2, _BN), lambda i: (0, 0, i)),
            pl.BlockSpec((1, _BN), lambda i: (0, i)),
            pl.BlockSpec((1, 2), lambda i: (0, 0)),
            pl.BlockSpec((4 * _H, 2 * _H), lambda i: (0, 0)),
            pl.BlockSpec((4 * _H, 2 * _H), lambda i: (0, 0)),
        ],
        out_specs=pl.BlockSpec((1, _GX * _GY, _H), lambda i: (i, 0, 0)),
        out_shape=jax.ShapeDtypeStruct((_NB, _GX * _GY, _H), jnp.float32),
        scratch_shapes=[
            pltpu.VMEM((_NH, 2 * _H, _BH), jnp.bfloat16),
            pltpu.VMEM((_NH, 2 * _H, _BH), jnp.float32),
            pltpu.VMEM((_NH, _H, _BH), jnp.float32),
        ],
        compiler_params=pltpu.CompilerParams(
            dimension_semantics=("arbitrary",),
        ),
        name="social_lstm_neighbors",
    )(xo, mask14, tgt14, wcat.astype(jnp.bfloat16), wcat)

    out = pl.pallas_call(
        _finish_kernel,
        out_shape=jax.ShapeDtypeStruct((1, 2), jnp.float32),
        name="social_lstm_finish",
    )(partials, tgt3, W_ih.T, W_hh.T, bsum, w1t3,
      b1.reshape(1, _H), W2.T, b2.reshape(1, _H), Wc.T, bc.reshape(1, 2))
    return out
